# Initial kernel scaffold; baseline (speedup 1.0000x reference)
import jax
import jax.numpy as jnp
from jax.experimental import pallas as pl


def kernel(x, edge_index, W0, att_src0, att_dst0, W1, att_src1, att_dst1):
    raise NotImplementedError("write your pallas kernel here")


# speedup vs baseline: 9.6144x; 9.6144x over previous
import jax
import jax.numpy as jnp
from jax import lax
from jax.experimental import pallas as pl
from jax.experimental.pallas import tpu as pltpu
from jax.experimental.pallas import tpu_sc as plsc

N = 10000
D = 128
DH = 32
NQ = D // DH
E = 320000
NC = 2
NS = 16
L = 16

EPT = E // NS
CH = 80
NCHUNK = EPT // CH
ROWS_MAIN = 632
ROWS_LAST = N - (NS - 1) * ROWS_MAIN
RED_W = 640
NPAD = NS * RED_W
RED_LAST = N - (NS - 1) * RED_W
EPS = 1e-16


def _dense_body(x_ref, w0_ref, s0_ref, d0_ref, w1_ref, s1_ref, d1_ref,
                *out_refs):
    hq_refs = out_refs[:NQ]
    as_ref, ad_ref = out_refs[NQ], out_refs[NQ + 1]
    x = x_ref[...]
    h0 = jnp.dot(x, w0_ref[...], preferred_element_type=jnp.float32)
    h1 = jnp.dot(x, w1_ref[...], preferred_element_type=jnp.float32)
    for q in range(NQ):
        hq_refs[q][0] = h0[:, q * DH:(q + 1) * DH]
        hq_refs[q][1] = h1[:, q * DH:(q + 1) * DH]
    as_ref[0] = jnp.sum(h0 * s0_ref[...][None, :], axis=1, keepdims=True)
    as_ref[1] = jnp.sum(h1 * s1_ref[...][None, :], axis=1, keepdims=True)
    ad_ref[0] = jnp.sum(h0 * d0_ref[...][None, :], axis=1, keepdims=True)
    ad_ref[1] = jnp.sum(h1 * d1_ref[...][None, :], axis=1, keepdims=True)


def _dense_stage(x, W0, a_s0, a_d0, W1, a_s1, a_d1):
    blk = 2000
    grid = N // blk
    full = lambda i: (0, 0)
    return pl.pallas_call(
        _dense_body,
        grid=(grid,),
        in_specs=[pl.BlockSpec((blk, D), lambda i: (i, 0))]
        + [pl.BlockSpec((D, D), full), pl.BlockSpec((D,), lambda i: (0,)),
           pl.BlockSpec((D,), lambda i: (0,))] * 2,
        out_specs=[pl.BlockSpec((2, blk, DH), lambda i: (0, i, 0))
                   for _ in range(NQ)]
        + [pl.BlockSpec((2, blk, 1), lambda i: (0, i, 0))] * 2,
        out_shape=[jax.ShapeDtypeStruct((2, N, DH), jnp.float32)
                   for _ in range(NQ)]
        + [jax.ShapeDtypeStruct((2, N, 1), jnp.float32),
           jax.ShapeDtypeStruct((2, N, 1), jnp.float32)],
    )(x, W0, a_s0, a_d0, W1, a_s1, a_d1)


def _sc_body(*refs):
    (src_hbm, dst_hbm) = refs[0:2]
    hq_hbm = refs[2:2 + NQ]
    (as_hbm, ad_hbm) = refs[2 + NQ:4 + NQ]
    featq = refs[4 + NQ:4 + 2 * NQ]
    (denom, ssum, cnt,
     asv, adv, pall, sidx, didx, rows, zbuf,
     denp, cntp, scop, dnf, redbuf,
     feat_sh, parts_sh, dnf_sh, sem) = refs[4 + 2 * NQ:]

    c = lax.axis_index("c")
    s = lax.axis_index("s")
    cN = c * N
    ebase = s * EPT
    zeros16 = jnp.zeros((L,), jnp.float32)
    ones16 = jnp.ones((L,), jnp.float32)

    pltpu.sync_copy(as_hbm.at[pl.ds(cN, N)], asv)
    pltpu.sync_copy(ad_hbm.at[pl.ds(cN, N)], adv)

    @pl.loop(0, CH)
    def _(i):
        for k in range(DH // L):
            zbuf[i, pl.ds(k * L, L)] = zeros16

    @pl.loop(0, NPAD // L)
    def _(i):
        denp[pl.ds(i * L, L)] = zeros16
        cntp[pl.ds(i * L, L)] = zeros16
        scop[pl.ds(i * L, L)] = zeros16

    row0 = s * ROWS_MAIN

    def zero_feat_rows(nrows):
        off = 0
        while nrows - off >= CH:
            pltpu.sync_copy(zbuf.at[pl.ds(0, CH)],
                            feat_sh.at[pl.ds(row0 + off, CH)])
            off += CH
        if nrows - off:
            pltpu.sync_copy(zbuf.at[pl.ds(0, nrows - off)],
                            feat_sh.at[pl.ds(row0 + off, nrows - off)])

    def zero_feat():
        @pl.when(s < NS - 1)
        def _():
            zero_feat_rows(ROWS_MAIN)

        @pl.when(s == NS - 1)
        def _():
            zero_feat_rows(ROWS_LAST)

    def dump_feat(out_hbm):
        @pl.when(s < NS - 1)
        def _():
            pltpu.sync_copy(feat_sh.at[pl.ds(row0, ROWS_MAIN)],
                            out_hbm.at[pl.ds(cN + row0, ROWS_MAIN)])

        @pl.when(s == NS - 1)
        def _():
            pltpu.sync_copy(feat_sh.at[pl.ds(row0, ROWS_LAST)],
                            out_hbm.at[pl.ds(cN + row0, ROWS_LAST)])

    def scale_rows_chunk(j):
        @pl.loop(0, CH // L)
        def _(g):
            p16 = pall[pl.ds(j * CH + g * L, L)]
            for lane in range(L):
                ps = p16[lane]
                e = g * L + lane
                for k in range(DH // L):
                    rows[e, pl.ds(k * L, L)] = rows[e, pl.ds(k * L, L)] * ps

    def feat_pass(h_hbm, out_hbm, first):
        zero_feat()
        plsc.subcore_barrier()

        @pl.loop(0, NCHUNK)
        def _(j):
            base = ebase + j * CH
            pltpu.sync_copy(src_hbm.at[pl.ds(base, CH)], sidx)
            pltpu.sync_copy(dst_hbm.at[pl.ds(base, CH)], didx.at[0])
            if first:
                for i in range(CH // L):
                    sv = sidx[pl.ds(i * L, L)]
                    dv = didx[0, pl.ds(i * L, L)]
                    a = (plsc.load_gather(asv, [sv])
                         + plsc.load_gather(adv, [dv]))
                    a = jnp.maximum(a, 0.2 * a)
                    p = jnp.exp(a)
                    pall[pl.ds(j * CH + i * L, L)] = p
                    plsc.addupdate_scatter(denp, [dv], p)
                    plsc.addupdate_scatter(cntp, [sv], ones16)
                    sidx[pl.ds(i * L, L)] = sv + cN
            else:
                for i in range(CH // L):
                    sv = sidx[pl.ds(i * L, L)]
                    sidx[pl.ds(i * L, L)] = sv + cN
            pltpu.async_copy(h_hbm.at[sidx], rows, sem).wait()
            scale_rows_chunk(j)
            pltpu.sync_copy(rows, feat_sh.at[didx.at[0]], add=True)

        plsc.subcore_barrier()
        dump_feat(out_hbm)
        plsc.subcore_barrier()

    for q in range(NQ):
        feat_pass(hq_hbm[q], featq[q], first=(q == 0))

    def reduce_all(stage, hbm_out, to_spmem):
        colbase = s * RED_W
        pltpu.sync_copy(parts_sh.at[:, pl.ds(colbase, RED_W)], redbuf)

        @pl.loop(0, RED_W // L)
        def _(g):
            acc = redbuf[0, pl.ds(g * L, L)]
            for r in range(1, NS):
                acc = acc + redbuf[r, pl.ds(g * L, L)]
            stage[pl.ds(colbase + g * L, L)] = acc

        @pl.when(s < NS - 1)
        def _():
            pltpu.sync_copy(stage.at[pl.ds(colbase, RED_W)],
                            hbm_out.at[pl.ds(cN + colbase, RED_W)])

        @pl.when(s == NS - 1)
        def _():
            pltpu.sync_copy(stage.at[pl.ds(colbase, RED_LAST)],
                            hbm_out.at[pl.ds(cN + colbase, RED_LAST)])

        if to_spmem:
            pltpu.sync_copy(stage.at[pl.ds(colbase, RED_W)],
                            dnf_sh.at[pl.ds(colbase, RED_W)])

    pltpu.sync_copy(denp, parts_sh.at[s])
    plsc.subcore_barrier()
    reduce_all(dnf, denom, True)
    plsc.subcore_barrier()
    pltpu.sync_copy(dnf_sh, dnf)

    @pl.loop(0, NCHUNK)
    def _(j):
        base = ebase + j * CH
        pltpu.sync_copy(src_hbm.at[pl.ds(base, CH)], sidx)
        pltpu.sync_copy(dst_hbm.at[pl.ds(base, CH)], didx.at[0])
        for i in range(CH // L):
            sv = sidx[pl.ds(i * L, L)]
            dv = didx[0, pl.ds(i * L, L)]
            p = pall[pl.ds(j * CH + i * L, L)]
            dn = plsc.load_gather(dnf, [dv])
            plsc.addupdate_scatter(scop, [sv], p / (dn + EPS))

    plsc.subcore_barrier()
    pltpu.sync_copy(scop, parts_sh.at[s])
    plsc.subcore_barrier()
    reduce_all(scop, ssum, False)

    plsc.subcore_barrier()
    pltpu.sync_copy(cntp, parts_sh.at[s])
    plsc.subcore_barrier()
    reduce_all(cntp, cnt, False)


def _sc_stage(src, dst, hq, as_cat, ad_cat):
    mesh = plsc.VectorSubcoreMesh(core_axis_name="c", subcore_axis_name="s",
                                  num_cores=NC, num_subcores=NS)
    f32 = jnp.float32
    kern = pl.kernel(
        _sc_body,
        out_type=[jax.ShapeDtypeStruct((2 * N, DH), f32)
                  for _ in range(NQ)]
        + [jax.ShapeDtypeStruct((2 * N,), f32),
           jax.ShapeDtypeStruct((2 * N,), f32),
           jax.ShapeDtypeStruct((2 * N,), f32)],
        mesh=mesh,
        compiler_params=pltpu.CompilerParams(needs_layout_passes=False,
                                             use_tc_tiling_on_sc=False),
        scratch_types=[
            pltpu.VMEM((N,), f32),
            pltpu.VMEM((N,), f32),
            pltpu.VMEM((EPT,), f32),
            pltpu.VMEM((CH,), jnp.int32),
            pltpu.VMEM((1, CH), jnp.int32),
            pltpu.VMEM((CH, DH), f32),
            pltpu.VMEM((CH, DH), f32),
            pltpu.VMEM((NPAD,), f32),
            pltpu.VMEM((NPAD,), f32),
            pltpu.VMEM((NPAD,), f32),
            pltpu.VMEM((NPAD,), f32),
            pltpu.VMEM((NS, RED_W), f32),
            pltpu.VMEM_SHARED((N, DH), f32),
            pltpu.VMEM_SHARED((NS, NPAD), f32),
            pltpu.VMEM_SHARED((NPAD,), f32),
            pltpu.SemaphoreType.DMA,
        ],
    )
    return kern(src, dst, *hq, as_cat, ad_cat)


def _final_body(*refs):
    fq0 = refs[0:NQ]
    fq1 = refs[NQ:2 * NQ]
    dn0_ref, dn1_ref, ss0_ref, ss1_ref, cnt_ref = refs[2 * NQ:2 * NQ + 5]
    feat_ref, score_ref = refs[2 * NQ + 5:]
    d0 = dn0_ref[...] + EPS
    d1 = dn1_ref[...] + EPS
    for q in range(NQ):
        f = fq0[q][...] / d0
        g = fq1[q][...] / d1
        feat_ref[:, pl.ds(q * DH, DH)] = (jnp.maximum(f, 0.01 * f)
                                          + jnp.maximum(g, 0.01 * g))
    ssum = ss0_ref[...] + ss1_ref[...]
    score_ref[...] = ssum / jnp.maximum(cnt_ref[...], 1.0)


def _final_stage(fq0, fq1, dn0, dn1, ss0, ss1, cntv):
    blk = 1000
    grid = N // blk
    half_spec = pl.BlockSpec((blk, DH), lambda i: (i, 0))
    col_spec = pl.BlockSpec((blk, 1), lambda i: (i, 0))
    return pl.pallas_call(
        _final_body,
        grid=(grid,),
        in_specs=[half_spec] * (2 * NQ) + [col_spec] * 5,
        out_specs=[pl.BlockSpec((blk, D), lambda i: (i, 0)), col_spec],
        out_shape=[
            jax.ShapeDtypeStruct((N, D), jnp.float32),
            jax.ShapeDtypeStruct((N, 1), jnp.float32),
        ],
    )(*fq0, *fq1, dn0, dn1, ss0, ss1, cntv)


@jax.jit
def kernel(x, edge_index, W0, att_src0, att_dst0, W1, att_src1, att_dst1):
    outs = _dense_stage(x, W0, att_src0, att_dst0, W1, att_src1, att_dst1)
    hq = [h.reshape(2 * N, DH) for h in outs[:NQ]]
    as_cat, ad_cat = outs[NQ], outs[NQ + 1]
    sc_outs = _sc_stage(edge_index[0], edge_index[1], hq,
                        as_cat.reshape(2 * N), ad_cat.reshape(2 * N))
    featq, denom, ssum, cnt = sc_outs[:NQ], sc_outs[NQ], sc_outs[NQ + 1], \
        sc_outs[NQ + 2]
    feature, score = _final_stage(
        [f[:N] for f in featq], [f[N:] for f in featq],
        denom[:N].reshape(N, 1), denom[N:].reshape(N, 1),
        ssum[:N].reshape(N, 1), ssum[N:].reshape(N, 1),
        cnt[:N].reshape(N, 1))
    return feature, score.reshape(N)


# speedup vs baseline: 24.3545x; 2.5331x over previous
import jax
import jax.numpy as jnp
from jax import lax
from jax.experimental import pallas as pl
from jax.experimental.pallas import tpu as pltpu
from jax.experimental.pallas import tpu_sc as plsc

N = 10000
D = 128
DH = 32
NQ = D // DH
E = 320000
NC = 2
NS = 16
L = 16

EPT = E // NS
CH = 80
NCHUNK = EPT // CH
SUP = 10
NSUP = NCHUNK // SUP
ROWS_MAIN = 632
ROWS_LAST = N - (NS - 1) * ROWS_MAIN
RED_W = 640
NPAD = NS * RED_W
RED_LAST = N - (NS - 1) * RED_W
EPS = 1e-16


def _dense_body(x_ref, w0_ref, s0_ref, d0_ref, w1_ref, s1_ref, d1_ref,
                *out_refs):
    hq_refs = out_refs[:NQ]
    as_ref, ad_ref = out_refs[NQ], out_refs[NQ + 1]
    x = x_ref[...]
    h0 = jnp.dot(x, w0_ref[...], preferred_element_type=jnp.float32)
    h1 = jnp.dot(x, w1_ref[...], preferred_element_type=jnp.float32)
    for q in range(NQ):
        hq_refs[q][0] = h0[:, q * DH:(q + 1) * DH]
        hq_refs[q][1] = h1[:, q * DH:(q + 1) * DH]
    as_ref[0] = jnp.sum(h0 * s0_ref[...][None, :], axis=1, keepdims=True)
    as_ref[1] = jnp.sum(h1 * s1_ref[...][None, :], axis=1, keepdims=True)
    ad_ref[0] = jnp.sum(h0 * d0_ref[...][None, :], axis=1, keepdims=True)
    ad_ref[1] = jnp.sum(h1 * d1_ref[...][None, :], axis=1, keepdims=True)


def _dense_stage(x, W0, a_s0, a_d0, W1, a_s1, a_d1):
    blk = 2000
    grid = N // blk
    full = lambda i: (0, 0)
    return pl.pallas_call(
        _dense_body,
        grid=(grid,),
        in_specs=[pl.BlockSpec((blk, D), lambda i: (i, 0))]
        + [pl.BlockSpec((D, D), full), pl.BlockSpec((D,), lambda i: (0,)),
           pl.BlockSpec((D,), lambda i: (0,))] * 2,
        out_specs=[pl.BlockSpec((2, blk, DH), lambda i: (0, i, 0))
                   for _ in range(NQ)]
        + [pl.BlockSpec((2, blk, 1), lambda i: (0, i, 0))] * 2,
        out_shape=[jax.ShapeDtypeStruct((2, N, DH), jnp.float32)
                   for _ in range(NQ)]
        + [jax.ShapeDtypeStruct((2, N, 1), jnp.float32),
           jax.ShapeDtypeStruct((2, N, 1), jnp.float32)],
    )(x, W0, a_s0, a_d0, W1, a_s1, a_d1)


def _sc_body(*refs):
    (src2_hbm, dst2_hbm) = refs[0:2]
    hq_hbm = refs[2:2 + NQ]
    (as_hbm, ad_hbm) = refs[2 + NQ:4 + NQ]
    featq = refs[4 + NQ:4 + 2 * NQ]
    (denom, ssum, cnt,
     asv, adv, pall, sidx, didx, rows0, rows1, zbuf,
     denp, cntp, scop, dnf, redbuf,
     feat_sh, parts_sh, sem0, sem1) = refs[4 + 2 * NQ:]

    rows_bufs = (rows0, rows1)
    sems = (sem0, sem1)
    c = lax.axis_index("c")
    s = lax.axis_index("s")
    cN = c * N
    chunk0 = s * NCHUNK
    zeros16 = jnp.zeros((L,), jnp.float32)
    ones16 = jnp.ones((L,), jnp.float32)

    pltpu.sync_copy(as_hbm.at[pl.ds(cN, N)], asv)
    pltpu.sync_copy(ad_hbm.at[pl.ds(cN, N)], adv)

    @pl.loop(0, CH)
    def _(i):
        for k in range(DH // L):
            zbuf[i, pl.ds(k * L, L)] = zeros16

    @pl.loop(0, NPAD // L)
    def _(i):
        denp[pl.ds(i * L, L)] = zeros16
        cntp[pl.ds(i * L, L)] = zeros16
        scop[pl.ds(i * L, L)] = zeros16

    row0 = s * ROWS_MAIN

    def zero_feat_rows(nrows):
        off = 0
        while nrows - off >= CH:
            pltpu.sync_copy(zbuf.at[pl.ds(0, CH)],
                            feat_sh.at[pl.ds(row0 + off, CH)])
            off += CH
        if nrows - off:
            pltpu.sync_copy(zbuf.at[pl.ds(0, nrows - off)],
                            feat_sh.at[pl.ds(row0 + off, nrows - off)])

    def zero_feat():
        @pl.when(s < NS - 1)
        def _():
            zero_feat_rows(ROWS_MAIN)

        @pl.when(s == NS - 1)
        def _():
            zero_feat_rows(ROWS_LAST)

    def dump_feat(out_hbm):
        @pl.when(s < NS - 1)
        def _():
            pltpu.sync_copy(feat_sh.at[pl.ds(row0, ROWS_MAIN)],
                            out_hbm.at[pl.ds(cN + row0, ROWS_MAIN)])

        @pl.when(s == NS - 1)
        def _():
            pltpu.sync_copy(feat_sh.at[pl.ds(row0, ROWS_LAST)],
                            out_hbm.at[pl.ds(cN + row0, ROWS_LAST)])

    def scale_rows(jj, rows):
        @pl.loop(0, CH // L)
        def _(g):
            p16 = pall[pl.ds(jj * CH + g * L, L)]
            for lane in range(L):
                ps = p16[lane]
                e = g * L + lane
                for k in range(DH // L):
                    rows[e, pl.ds(k * L, L)] = rows[e, pl.ds(k * L, L)] * ps

    def feat_pass(h_hbm, out_hbm, q):
        zero_feat()
        plsc.subcore_barrier()

        @pl.loop(0, NSUP)
        def _(u):
            jbase = u * SUP
            pltpu.sync_copy(src2_hbm.at[pl.ds(chunk0 + jbase, SUP)], sidx)
            pltpu.sync_copy(dst2_hbm.at[pl.ds(chunk0 + jbase, SUP)], didx)
            descs = [None, None]

            def prep_chunk(k):
                if q == 0:
                    for i in range(CH // L):
                        sv = sidx[k, pl.ds(i * L, L)]
                        dv = didx[k, pl.ds(i * L, L)]
                        a = (plsc.load_gather(asv, [sv])
                             + plsc.load_gather(adv, [dv]))
                        a = jnp.maximum(a, 0.2 * a)
                        p = jnp.exp(a)
                        pall[pl.ds((jbase + k) * CH + i * L, L)] = p
                        plsc.addupdate_scatter(denp, [dv], p)
                        plsc.addupdate_scatter(cntp, [sv], ones16)
                        sidx[k, pl.ds(i * L, L)] = sv + cN
                else:
                    for i in range(CH // L):
                        sv = sidx[k, pl.ds(i * L, L)]
                        if q == 1:
                            dv = didx[k, pl.ds(i * L, L)]
                            p = pall[pl.ds((jbase + k) * CH + i * L, L)]
                            dn = plsc.load_gather(dnf, [dv])
                            plsc.addupdate_scatter(scop, [sv],
                                                   p / (dn + EPS))
                        sidx[k, pl.ds(i * L, L)] = sv + cN

            def drain_chunk(k):
                descs[k % 2].wait()
                scale_rows(jbase + k, rows_bufs[k % 2])
                pltpu.sync_copy(rows_bufs[k % 2],
                                feat_sh.at[didx.at[k]], add=True)

            for k in range(SUP):
                prep_chunk(k)
                descs[k % 2] = pltpu.async_copy(
                    h_hbm.at[sidx.at[k]], rows_bufs[k % 2], sems[k % 2])
                if k > 0:
                    drain_chunk(k - 1)
            drain_chunk(SUP - 1)

        plsc.subcore_barrier()
        dump_feat(out_hbm)
        plsc.subcore_barrier()

    def reduce_all(stage, hbm_out):
        colbase = s * RED_W
        pltpu.sync_copy(parts_sh.at[:, pl.ds(colbase, RED_W)], redbuf)

        @pl.loop(0, RED_W // L)
        def _(g):
            acc = redbuf[0, pl.ds(g * L, L)]
            for r in range(1, NS):
                acc = acc + redbuf[r, pl.ds(g * L, L)]
            stage[pl.ds(colbase + g * L, L)] = acc

        @pl.when(s < NS - 1)
        def _():
            pltpu.sync_copy(stage.at[pl.ds(colbase, RED_W)],
                            hbm_out.at[pl.ds(cN + colbase, RED_W)])

        @pl.when(s == NS - 1)
        def _():
            pltpu.sync_copy(stage.at[pl.ds(colbase, RED_LAST)],
                            hbm_out.at[pl.ds(cN + colbase, RED_LAST)])

    feat_pass(hq_hbm[0], featq[0], 0)

    pltpu.sync_copy(denp, parts_sh.at[s])
    plsc.subcore_barrier()
    reduce_all(dnf, denom)
    plsc.subcore_barrier()
    pltpu.sync_copy(denom.at[pl.ds(cN, N)], dnf.at[pl.ds(0, N)])

    for q in range(1, NQ):
        feat_pass(hq_hbm[q], featq[q], q)

    plsc.subcore_barrier()
    pltpu.sync_copy(scop, parts_sh.at[s])
    plsc.subcore_barrier()
    reduce_all(scop, ssum)

    plsc.subcore_barrier()
    pltpu.sync_copy(cntp, parts_sh.at[s])
    plsc.subcore_barrier()
    reduce_all(cntp, cnt)


def _sc_stage(src2, dst2, hq, as_cat, ad_cat):
    mesh = plsc.VectorSubcoreMesh(core_axis_name="c", subcore_axis_name="s",
                                  num_cores=NC, num_subcores=NS)
    f32 = jnp.float32
    kern = pl.kernel(
        _sc_body,
        out_type=[jax.ShapeDtypeStruct((2 * N, DH), f32)
                  for _ in range(NQ)]
        + [jax.ShapeDtypeStruct((2 * N,), f32),
           jax.ShapeDtypeStruct((2 * N,), f32),
           jax.ShapeDtypeStruct((2 * N,), f32)],
        mesh=mesh,
        compiler_params=pltpu.CompilerParams(needs_layout_passes=False,
                                             use_tc_tiling_on_sc=False),
        scratch_types=[
            pltpu.VMEM((N,), f32),
            pltpu.VMEM((N,), f32),
            pltpu.VMEM((EPT,), f32),
            pltpu.VMEM((SUP, CH), jnp.int32),
            pltpu.VMEM((SUP, CH), jnp.int32),
            pltpu.VMEM((CH, DH), f32),
            pltpu.VMEM((CH, DH), f32),
            pltpu.VMEM((CH, DH), f32),
            pltpu.VMEM((NPAD,), f32),
            pltpu.VMEM((NPAD,), f32),
            pltpu.VMEM((NPAD,), f32),
            pltpu.VMEM((NPAD,), f32),
            pltpu.VMEM((NS, RED_W), f32),
            pltpu.VMEM_SHARED((N, DH), f32),
            pltpu.VMEM_SHARED((NS, NPAD), f32),
            pltpu.SemaphoreType.DMA,
            pltpu.SemaphoreType.DMA,
        ],
    )
    return kern(src2, dst2, *hq, as_cat, ad_cat)


def _final_body(*refs):
    fq0 = refs[0:NQ]
    fq1 = refs[NQ:2 * NQ]
    dn0_ref, dn1_ref, ss0_ref, ss1_ref, cnt_ref = refs[2 * NQ:2 * NQ + 5]
    feat_ref, score_ref = refs[2 * NQ + 5:]
    d0 = dn0_ref[...] + EPS
    d1 = dn1_ref[...] + EPS
    for q in range(NQ):
        f = fq0[q][...] / d0
        g = fq1[q][...] / d1
        feat_ref[:, pl.ds(q * DH, DH)] = (jnp.maximum(f, 0.01 * f)
                                          + jnp.maximum(g, 0.01 * g))
    ssum = ss0_ref[...] + ss1_ref[...]
    score_ref[...] = ssum / jnp.maximum(cnt_ref[...], 1.0)


def _final_stage(fq0, fq1, dn0, dn1, ss0, ss1, cntv):
    blk = 1000
    grid = N // blk
    half_spec = pl.BlockSpec((blk, DH), lambda i: (i, 0))
    col_spec = pl.BlockSpec((blk, 1), lambda i: (i, 0))
    return pl.pallas_call(
        _final_body,
        grid=(grid,),
        in_specs=[half_spec] * (2 * NQ) + [col_spec] * 5,
        out_specs=[pl.BlockSpec((blk, D), lambda i: (i, 0)), col_spec],
        out_shape=[
            jax.ShapeDtypeStruct((N, D), jnp.float32),
            jax.ShapeDtypeStruct((N, 1), jnp.float32),
        ],
    )(*fq0, *fq1, dn0, dn1, ss0, ss1, cntv)


@jax.jit
def kernel(x, edge_index, W0, att_src0, att_dst0, W1, att_src1, att_dst1):
    outs = _dense_stage(x, W0, att_src0, att_dst0, W1, att_src1, att_dst1)
    hq = [h.reshape(2 * N, DH) for h in outs[:NQ]]
    as_cat, ad_cat = outs[NQ], outs[NQ + 1]
    src2 = edge_index[0].reshape(E // CH, CH)
    dst2 = edge_index[1].reshape(E // CH, CH)
    sc_outs = _sc_stage(src2, dst2, hq,
                        as_cat.reshape(2 * N), ad_cat.reshape(2 * N))
    featq, denom, ssum, cnt = sc_outs[:NQ], sc_outs[NQ], sc_outs[NQ + 1], \
        sc_outs[NQ + 2]
    feature, score = _final_stage(
        [f[:N] for f in featq], [f[N:] for f in featq],
        denom[:N].reshape(N, 1), denom[N:].reshape(N, 1),
        ssum[:N].reshape(N, 1), ssum[N:].reshape(N, 1),
        cnt[:N].reshape(N, 1))
    return feature, score.reshape(N)


# speedup vs baseline: 28.1130x; 1.1543x over previous
import jax
import jax.numpy as jnp
from jax import lax
from jax.experimental import pallas as pl
from jax.experimental.pallas import tpu as pltpu
from jax.experimental.pallas import tpu_sc as plsc

N = 10000
D = 128
DH = 32
NQ = D // DH
E = 320000
NC = 2
NS = 16
L = 16

EPT = E // NS
CH = 80
NCHUNK = EPT // CH
SUP = 10
NSUP = NCHUNK // SUP
NBUF = 4
ROWS_MAIN = 632
ROWS_LAST = N - (NS - 1) * ROWS_MAIN
RED_W = 640
NPAD = NS * RED_W
NPAD2 = NPAD // 2
RED_LAST = N - (NS - 1) * RED_W
EPS = 1e-16


def _dense_body(x_ref, w0_ref, s0_ref, d0_ref, w1_ref, s1_ref, d1_ref,
                h_ref, as_ref, ad_ref):
    x = x_ref[...]
    h0 = jnp.dot(x, w0_ref[...], preferred_element_type=jnp.float32)
    h1 = jnp.dot(x, w1_ref[...], preferred_element_type=jnp.float32)
    for q in range(NQ):
        h_ref[q, 0] = h0[:, q * DH:(q + 1) * DH]
        h_ref[q, 1] = h1[:, q * DH:(q + 1) * DH]
    as_ref[0] = jnp.sum(h0 * s0_ref[...][None, :], axis=1, keepdims=True)
    as_ref[1] = jnp.sum(h1 * s1_ref[...][None, :], axis=1, keepdims=True)
    ad_ref[0] = jnp.sum(h0 * d0_ref[...][None, :], axis=1, keepdims=True)
    ad_ref[1] = jnp.sum(h1 * d1_ref[...][None, :], axis=1, keepdims=True)


def _dense_stage(x, W0, a_s0, a_d0, W1, a_s1, a_d1):
    blk = 2000
    grid = N // blk
    full = lambda i: (0, 0)
    return pl.pallas_call(
        _dense_body,
        grid=(grid,),
        in_specs=[pl.BlockSpec((blk, D), lambda i: (i, 0))]
        + [pl.BlockSpec((D, D), full), pl.BlockSpec((D,), lambda i: (0,)),
           pl.BlockSpec((D,), lambda i: (0,))] * 2,
        out_specs=[pl.BlockSpec((NQ, 2, blk, DH), lambda i: (0, 0, i, 0))]
        + [pl.BlockSpec((2, blk, 1), lambda i: (0, i, 0))] * 2,
        out_shape=[jax.ShapeDtypeStruct((NQ, 2, N, DH), jnp.float32),
                   jax.ShapeDtypeStruct((2, N, 1), jnp.float32),
                   jax.ShapeDtypeStruct((2, N, 1), jnp.float32)],
    )(x, W0, a_s0, a_d0, W1, a_s1, a_d1)


def _sc_body(src2_hbm, dst2_hbm, h_hbm, as_hbm, ad_hbm,
             feat_out, denom, ssum, cnt,
             asv, adv, pall, sidx, didx,
             rows0, rows1, rows2, rows3, zbuf,
             denp, cntp, scop, dnf, redbuf,
             feat_sh, parts_sh,
             gs0, gs1, gs2, gs3, ss0, ss1, ss2, ss3):
    rows_bufs = (rows0, rows1, rows2, rows3)
    gsems = (gs0, gs1, gs2, gs3)
    ssems = (ss0, ss1, ss2, ss3)
    c = lax.axis_index("c")
    s = lax.axis_index("s")
    cN = c * N
    chunk0 = s * NCHUNK
    zeros16 = jnp.zeros((L,), jnp.float32)
    ones16 = jnp.ones((L,), jnp.float32)
    zeros16i = jnp.zeros((L,), jnp.int32)

    pltpu.sync_copy(as_hbm.at[pl.ds(cN, N)], asv)
    pltpu.sync_copy(ad_hbm.at[pl.ds(cN, N)], adv)

    @pl.loop(0, CH)
    def _(i):
        for k in range(DH // L):
            zbuf[i, pl.ds(k * L, L)] = zeros16

    @pl.loop(0, NPAD // L)
    def _(i):
        denp[pl.ds(i * L, L)] = zeros16
        cntp[pl.ds(i * L, L)] = zeros16
        scop[pl.ds(i * L, L)] = zeros16

    @pl.loop(0, SUP)
    def _(k):
        for i in range(CH // L):
            didx[k, pl.ds(i * L, L)] = zeros16i

    row0 = s * ROWS_MAIN

    def zero_feat_rows(nrows):
        off = 0
        while nrows - off >= CH:
            pltpu.sync_copy(zbuf.at[pl.ds(0, CH)],
                            feat_sh.at[pl.ds(row0 + off, CH)])
            off += CH
        if nrows - off:
            pltpu.sync_copy(zbuf.at[pl.ds(0, nrows - off)],
                            feat_sh.at[pl.ds(row0 + off, nrows - off)])

    def zero_feat():
        @pl.when(s < NS - 1)
        def _():
            zero_feat_rows(ROWS_MAIN)

        @pl.when(s == NS - 1)
        def _():
            zero_feat_rows(ROWS_LAST)

    def dump_feat(qd):
        obase = qd * (2 * N) + cN + row0

        @pl.when(s < NS - 1)
        def _():
            pltpu.sync_copy(feat_sh.at[pl.ds(row0, ROWS_MAIN)],
                            feat_out.at[pl.ds(obase, ROWS_MAIN)])

        @pl.when(s == NS - 1)
        def _():
            pltpu.sync_copy(feat_sh.at[pl.ds(row0, ROWS_LAST)],
                            feat_out.at[pl.ds(obase, ROWS_LAST)])

    def scale_rows(jj, rows):
        @pl.loop(0, CH // L)
        def _(g):
            p16 = pall[pl.ds(jj * CH + g * L, L)]
            for lane in range(L):
                ps = p16[lane]
                e = g * L + lane
                for k in range(DH // L):
                    rows[e, pl.ds(k * L, L)] = rows[e, pl.ds(k * L, L)] * ps

    def reduce_all(buf, hbm_out):
        colbase = s * RED_W

        for r in range(2):
            pltpu.sync_copy(buf.at[pl.ds(r * NPAD2, NPAD2)],
                            parts_sh.at[s])
            plsc.subcore_barrier()

            @pl.when((s // 8) == r)
            def _(r=r):
                pltpu.sync_copy(
                    parts_sh.at[:, pl.ds((s % 8) * RED_W, RED_W)], redbuf)

                @pl.loop(0, RED_W // L)
                def _(g):
                    acc = redbuf[0, pl.ds(g * L, L)]
                    for t in range(1, NS):
                        acc = acc + redbuf[t, pl.ds(g * L, L)]
                    buf[pl.ds(colbase + g * L, L)] = acc

            plsc.subcore_barrier()

        @pl.when(s < NS - 1)
        def _():
            pltpu.sync_copy(buf.at[pl.ds(colbase, RED_W)],
                            hbm_out.at[pl.ds(cN + colbase, RED_W)])

        @pl.when(s == NS - 1)
        def _():
            pltpu.sync_copy(buf.at[pl.ds(colbase, RED_LAST)],
                            hbm_out.at[pl.ds(cN + colbase, RED_LAST)])

    @pl.loop(0, NQ)
    def _(qd):
        @pl.when(qd == 1)
        def _():
            reduce_all(denp, denom)
            plsc.subcore_barrier()
            pltpu.sync_copy(denom.at[pl.ds(cN, N)], dnf.at[pl.ds(0, N)])

        zero_feat()
        plsc.subcore_barrier()

        for b in range(NBUF):
            pltpu.async_copy(zbuf, feat_sh.at[didx.at[0]], ssems[b],
                             add=True)

        hbase = qd * (2 * N) + cN

        @pl.loop(0, NSUP)
        def _(u):
            jbase = u * SUP
            pltpu.sync_copy(src2_hbm.at[pl.ds(chunk0 + jbase, SUP)], sidx)
            pltpu.sync_copy(dst2_hbm.at[pl.ds(chunk0 + jbase, SUP)], didx)
            gdescs = [None] * NBUF

            for step in range(SUP + 2):
                k = step
                if k < SUP:
                    for i in range(CH // L):
                        sv = sidx[k, pl.ds(i * L, L)]
                        dv = didx[k, pl.ds(i * L, L)]

                        @pl.when(qd == 0)
                        def _(sv=sv, dv=dv, k=k, i=i):
                            a = (plsc.load_gather(asv, [sv])
                                 + plsc.load_gather(adv, [dv]))
                            a = jnp.maximum(a, 0.2 * a)
                            p = jnp.exp(a)
                            pall[pl.ds((jbase + k) * CH + i * L, L)] = p
                            plsc.addupdate_scatter(denp, [dv], p)
                            plsc.addupdate_scatter(cntp, [sv], ones16)

                        @pl.when(qd == 1)
                        def _(sv=sv, dv=dv, k=k, i=i):
                            p = pall[pl.ds((jbase + k) * CH + i * L, L)]
                            dn = plsc.load_gather(dnf, [dv])
                            plsc.addupdate_scatter(scop, [sv],
                                                   p / (dn + EPS))

                        sidx[k, pl.ds(i * L, L)] = sv + hbase
                    b = k % NBUF
                    pltpu.make_async_copy(rows_bufs[b],
                                          feat_sh.at[didx.at[0]],
                                          ssems[b]).wait()
                    gdescs[b] = pltpu.async_copy(
                        h_hbm.at[sidx.at[k]], rows_bufs[b], gsems[b])
                if 0 <= k - 2 < SUP:
                    b2 = (k - 2) % NBUF
                    gdescs[b2].wait()
                    scale_rows(jbase + (k - 2), rows_bufs[b2])
                    pltpu.async_copy(rows_bufs[b2],
                                     feat_sh.at[didx.at[k - 2]],
                                     ssems[b2], add=True)

        for b in range(NBUF):
            pltpu.make_async_copy(rows_bufs[b], feat_sh.at[didx.at[0]],
                                  ssems[b]).wait()

        plsc.subcore_barrier()
        dump_feat(qd)
        plsc.subcore_barrier()

    reduce_all(scop, ssum)
    reduce_all(cntp, cnt)


def _sc_stage(src2, dst2, h_all, as_cat, ad_cat):
    mesh = plsc.VectorSubcoreMesh(core_axis_name="c", subcore_axis_name="s",
                                  num_cores=NC, num_subcores=NS)
    f32 = jnp.float32
    kern = pl.kernel(
        _sc_body,
        out_type=[
            jax.ShapeDtypeStruct((NQ * 2 * N, DH), f32),
            jax.ShapeDtypeStruct((2 * N,), f32),
            jax.ShapeDtypeStruct((2 * N,), f32),
            jax.ShapeDtypeStruct((2 * N,), f32),
        ],
        mesh=mesh,
        compiler_params=pltpu.CompilerParams(needs_layout_passes=False,
                                             use_tc_tiling_on_sc=False),
        scratch_types=[
            pltpu.VMEM((N,), f32),
            pltpu.VMEM((N,), f32),
            pltpu.VMEM((EPT,), f32),
            pltpu.VMEM((SUP, CH), jnp.int32),
            pltpu.VMEM((SUP, CH), jnp.int32),
            pltpu.VMEM((CH, DH), f32),
            pltpu.VMEM((CH, DH), f32),
            pltpu.VMEM((CH, DH), f32),
            pltpu.VMEM((CH, DH), f32),
            pltpu.VMEM((CH, DH), f32),
            pltpu.VMEM((NPAD,), f32),
            pltpu.VMEM((NPAD,), f32),
            pltpu.VMEM((NPAD,), f32),
            pltpu.VMEM((NPAD,), f32),
            pltpu.VMEM((NS, RED_W), f32),
            pltpu.VMEM_SHARED((N, DH), f32),
            pltpu.VMEM_SHARED((NS, NPAD2), f32),
            pltpu.SemaphoreType.DMA,
            pltpu.SemaphoreType.DMA,
            pltpu.SemaphoreType.DMA,
            pltpu.SemaphoreType.DMA,
            pltpu.SemaphoreType.DMA,
            pltpu.SemaphoreType.DMA,
            pltpu.SemaphoreType.DMA,
            pltpu.SemaphoreType.DMA,
        ],
    )
    return kern(src2, dst2, h_all, as_cat, ad_cat)


def _final_body(*refs):
    fq0 = refs[0:NQ]
    fq1 = refs[NQ:2 * NQ]
    dn0_ref, dn1_ref, ss0_ref, ss1_ref, cnt_ref = refs[2 * NQ:2 * NQ + 5]
    feat_ref, score_ref = refs[2 * NQ + 5:]
    d0 = dn0_ref[...] + EPS
    d1 = dn1_ref[...] + EPS
    for q in range(NQ):
        f = fq0[q][...] / d0
        g = fq1[q][...] / d1
        feat_ref[:, pl.ds(q * DH, DH)] = (jnp.maximum(f, 0.01 * f)
                                          + jnp.maximum(g, 0.01 * g))
    ssum = ss0_ref[...] + ss1_ref[...]
    score_ref[...] = ssum / jnp.maximum(cnt_ref[...], 1.0)


def _final_stage(fq0, fq1, dn0, dn1, ss0, ss1, cntv):
    blk = 1000
    grid = N // blk
    half_spec = pl.BlockSpec((blk, DH), lambda i: (i, 0))
    col_spec = pl.BlockSpec((blk, 1), lambda i: (i, 0))
    return pl.pallas_call(
        _final_body,
        grid=(grid,),
        in_specs=[half_spec] * (2 * NQ) + [col_spec] * 5,
        out_specs=[pl.BlockSpec((blk, D), lambda i: (i, 0)), col_spec],
        out_shape=[
            jax.ShapeDtypeStruct((N, D), jnp.float32),
            jax.ShapeDtypeStruct((N, 1), jnp.float32),
        ],
    )(*fq0, *fq1, dn0, dn1, ss0, ss1, cntv)


@jax.jit
def kernel(x, edge_index, W0, att_src0, att_dst0, W1, att_src1, att_dst1):
    h_all, as_cat, ad_cat = _dense_stage(x, W0, att_src0, att_dst0,
                                         W1, att_src1, att_dst1)
    src2 = edge_index[0].reshape(E // CH, CH)
    dst2 = edge_index[1].reshape(E // CH, CH)
    sc_outs = _sc_stage(src2, dst2, h_all.reshape(NQ * 2 * N, DH),
                        as_cat.reshape(2 * N), ad_cat.reshape(2 * N))
    feat_all, denom, ssum, cnt = sc_outs
    fq0 = [feat_all[q * 2 * N:q * 2 * N + N] for q in range(NQ)]
    fq1 = [feat_all[q * 2 * N + N:(q + 1) * 2 * N] for q in range(NQ)]
    feature, score = _final_stage(
        fq0, fq1,
        denom[:N].reshape(N, 1), denom[N:].reshape(N, 1),
        ssum[:N].reshape(N, 1), ssum[N:].reshape(N, 1),
        cnt[:N].reshape(N, 1))
    return feature, score.reshape(N)


# speedup vs baseline: 31.2505x; 1.1116x over previous
import jax
import jax.numpy as jnp
from jax import lax
from jax.experimental import pallas as pl
from jax.experimental.pallas import tpu as pltpu
from jax.experimental.pallas import tpu_sc as plsc

N = 10000
D = 128
DH = 32
NQ = D // DH
E = 320000
NC = 2
NS = 16
L = 16

EPT = E // NS
CH = 80
NCHUNK = EPT // CH
SUP = 5
PAIR = 2 * SUP
NITER = NCHUNK // PAIR
NBUF = 4
ROWS_MAIN = 632
ROWS_LAST = N - (NS - 1) * ROWS_MAIN
RED_W = 640
NPAD = NS * RED_W
NPAD2 = NPAD // 2
RED_LAST = N - (NS - 1) * RED_W
EPS = 1e-16


def _dense_body(x_ref, w0_ref, s0_ref, d0_ref, w1_ref, s1_ref, d1_ref,
                h_ref, as_ref, ad_ref):
    x = x_ref[...]
    h0 = jnp.dot(x, w0_ref[...], preferred_element_type=jnp.float32)
    h1 = jnp.dot(x, w1_ref[...], preferred_element_type=jnp.float32)
    for q in range(NQ):
        h_ref[q, 0] = h0[:, q * DH:(q + 1) * DH]
        h_ref[q, 1] = h1[:, q * DH:(q + 1) * DH]
    as_ref[0] = jnp.sum(h0 * s0_ref[...][None, :], axis=1, keepdims=True)
    as_ref[1] = jnp.sum(h1 * s1_ref[...][None, :], axis=1, keepdims=True)
    ad_ref[0] = jnp.sum(h0 * d0_ref[...][None, :], axis=1, keepdims=True)
    ad_ref[1] = jnp.sum(h1 * d1_ref[...][None, :], axis=1, keepdims=True)


def _dense_stage(x, W0, a_s0, a_d0, W1, a_s1, a_d1):
    blk = 2000
    grid = N // blk
    full = lambda i: (0, 0)
    return pl.pallas_call(
        _dense_body,
        grid=(grid,),
        in_specs=[pl.BlockSpec((blk, D), lambda i: (i, 0))]
        + [pl.BlockSpec((D, D), full), pl.BlockSpec((D,), lambda i: (0,)),
           pl.BlockSpec((D,), lambda i: (0,))] * 2,
        out_specs=[pl.BlockSpec((NQ, 2, blk, DH), lambda i: (0, 0, i, 0))]
        + [pl.BlockSpec((2, blk, 1), lambda i: (0, i, 0))] * 2,
        out_shape=[jax.ShapeDtypeStruct((NQ, 2, N, DH), jnp.float32),
                   jax.ShapeDtypeStruct((2, N, 1), jnp.float32),
                   jax.ShapeDtypeStruct((2, N, 1), jnp.float32)],
    )(x, W0, a_s0, a_d0, W1, a_s1, a_d1)


def _sc_body(src2_hbm, dst2_hbm, h_hbm, as_hbm, ad_hbm,
             feat_out, denom, ssum, cnt,
             asv, adv, pall, sidx, didx,
             rows0, rows1, rows2, rows3, zbuf,
             denp, cntp, scop, dnf, redbuf,
             feat_sh, parts_sh,
             gs0, gs1, gs2, gs3, ss0, ss1, ss2, ss3, is0, is1):
    rows_bufs = (rows0, rows1, rows2, rows3)
    gsems = (gs0, gs1, gs2, gs3)
    ssems = (ss0, ss1, ss2, ss3)
    isems = (is0, is1)
    c = lax.axis_index("c")
    s = lax.axis_index("s")
    cN = c * N
    chunk0 = s * NCHUNK
    zeros16 = jnp.zeros((L,), jnp.float32)
    ones16 = jnp.ones((L,), jnp.float32)

    pltpu.sync_copy(as_hbm.at[pl.ds(cN, N)], asv)
    pltpu.sync_copy(ad_hbm.at[pl.ds(cN, N)], adv)

    @pl.loop(0, CH)
    def _(i):
        for k in range(DH // L):
            zbuf[i, pl.ds(k * L, L)] = zeros16

    @pl.loop(0, NPAD // L)
    def _(i):
        denp[pl.ds(i * L, L)] = zeros16
        cntp[pl.ds(i * L, L)] = zeros16
        scop[pl.ds(i * L, L)] = zeros16

    row0 = s * ROWS_MAIN

    def zero_feat_rows(nrows):
        off = 0
        while nrows - off >= CH:
            pltpu.sync_copy(zbuf.at[pl.ds(0, CH)],
                            feat_sh.at[pl.ds(row0 + off, CH)])
            off += CH
        if nrows - off:
            pltpu.sync_copy(zbuf.at[pl.ds(0, nrows - off)],
                            feat_sh.at[pl.ds(row0 + off, nrows - off)])

    def zero_feat():
        @pl.when(s < NS - 1)
        def _():
            zero_feat_rows(ROWS_MAIN)

        @pl.when(s == NS - 1)
        def _():
            zero_feat_rows(ROWS_LAST)

    def dump_feat(qd):
        obase = qd * (2 * N) + cN + row0

        @pl.when(s < NS - 1)
        def _():
            pltpu.sync_copy(feat_sh.at[pl.ds(row0, ROWS_MAIN)],
                            feat_out.at[pl.ds(obase, ROWS_MAIN)])

        @pl.when(s == NS - 1)
        def _():
            pltpu.sync_copy(feat_sh.at[pl.ds(row0, ROWS_LAST)],
                            feat_out.at[pl.ds(obase, ROWS_LAST)])

    def scale_rows(jj, rows):
        @pl.loop(0, CH // L)
        def _(g):
            p16 = pall[pl.ds(jj * CH + g * L, L)]
            for lane in range(L):
                ps = p16[lane]
                e = g * L + lane
                for k in range(DH // L):
                    rows[e, pl.ds(k * L, L)] = rows[e, pl.ds(k * L, L)] * ps

    def reduce_all(buf, hbm_out):
        colbase = s * RED_W

        for r in range(2):
            pltpu.sync_copy(buf.at[pl.ds(r * NPAD2, NPAD2)],
                            parts_sh.at[s])
            plsc.subcore_barrier()

            @pl.when((s // 8) == r)
            def _(r=r):
                pltpu.sync_copy(
                    parts_sh.at[:, pl.ds((s % 8) * RED_W, RED_W)], redbuf)

                @pl.loop(0, RED_W // L)
                def _(g):
                    acc = redbuf[0, pl.ds(g * L, L)]
                    for t in range(1, NS):
                        acc = acc + redbuf[t, pl.ds(g * L, L)]
                    buf[pl.ds(colbase + g * L, L)] = acc

            plsc.subcore_barrier()

        @pl.when(s < NS - 1)
        def _():
            pltpu.sync_copy(buf.at[pl.ds(colbase, RED_W)],
                            hbm_out.at[pl.ds(cN + colbase, RED_W)])

        @pl.when(s == NS - 1)
        def _():
            pltpu.sync_copy(buf.at[pl.ds(colbase, RED_LAST)],
                            hbm_out.at[pl.ds(cN + colbase, RED_LAST)])

    @pl.loop(0, NQ)
    def _(qd):
        @pl.when(qd == 1)
        def _():
            reduce_all(denp, denom)
            plsc.subcore_barrier()
            pltpu.sync_copy(denom.at[pl.ds(cN, N)], dnf.at[pl.ds(0, N)])

        zero_feat()
        plsc.subcore_barrier()

        hbase = qd * (2 * N) + cN

        def issue_idx(buf, rowoff):
            pltpu.async_copy(src2_hbm.at[pl.ds(rowoff, SUP)],
                             sidx.at[buf], isems[buf])
            pltpu.async_copy(dst2_hbm.at[pl.ds(rowoff, SUP)],
                             didx.at[buf], isems[buf])

        def wait_idx(buf):
            for _ in range(2):
                pltpu.make_async_copy(src2_hbm.at[pl.ds(chunk0, SUP)],
                                      sidx.at[buf], isems[buf]).wait()

        issue_idx(0, chunk0)
        issue_idx(1, chunk0 + SUP)

        @pl.loop(0, NITER)
        def _(u):
            jbase = u * PAIR
            nextoff = chunk0 + ((u + 1) % NITER) * PAIR
            gdescs = [None] * NBUF

            for step in range(PAIR + 2):
                k = step
                if k < PAIR:
                    half, kk = divmod(k, SUP)
                    if kk == 0:
                        wait_idx(half)
                    for i in range(CH // L):
                        sv = sidx[half, kk, pl.ds(i * L, L)]
                        dv = didx[half, kk, pl.ds(i * L, L)]

                        @pl.when(qd == 0)
                        def _(sv=sv, dv=dv, k=k, i=i):
                            a = (plsc.load_gather(asv, [sv])
                                 + plsc.load_gather(adv, [dv]))
                            a = jnp.maximum(a, 0.2 * a)
                            p = jnp.exp(a)
                            pall[pl.ds((jbase + k) * CH + i * L, L)] = p
                            plsc.addupdate_scatter(denp, [dv], p)
                            plsc.addupdate_scatter(cntp, [sv], ones16)

                        @pl.when(qd == 1)
                        def _(sv=sv, dv=dv, k=k, i=i):
                            p = pall[pl.ds((jbase + k) * CH + i * L, L)]
                            dn = plsc.load_gather(dnf, [dv])
                            plsc.addupdate_scatter(scop, [sv],
                                                   p / (dn + EPS))

                        sidx[half, kk, pl.ds(i * L, L)] = sv + hbase
                    b = k % NBUF
                    if k >= NBUF:
                        pltpu.make_async_copy(rows_bufs[b],
                                              feat_sh.at[didx.at[0, 0]],
                                              ssems[b]).wait()
                    gdescs[b] = pltpu.async_copy(
                        h_hbm.at[sidx.at[half, kk]], rows_bufs[b],
                        gsems[b])
                    if k == 2 * NBUF:
                        issue_idx(0, nextoff)
                if 0 <= k - 2 < PAIR:
                    h2, kk2 = divmod(k - 2, SUP)
                    b2 = (k - 2) % NBUF
                    gdescs[b2].wait()
                    scale_rows(jbase + (k - 2), rows_bufs[b2])
                    pltpu.async_copy(rows_bufs[b2],
                                     feat_sh.at[didx.at[h2, kk2]],
                                     ssems[b2], add=True)

            for b in range(NBUF):
                pltpu.make_async_copy(rows_bufs[b],
                                      feat_sh.at[didx.at[0, 0]],
                                      ssems[b]).wait()
            issue_idx(1, nextoff + SUP)

        wait_idx(0)
        wait_idx(1)

        plsc.subcore_barrier()
        dump_feat(qd)
        plsc.subcore_barrier()

    reduce_all(scop, ssum)
    reduce_all(cntp, cnt)


def _sc_stage(src2, dst2, h_all, as_cat, ad_cat):
    mesh = plsc.VectorSubcoreMesh(core_axis_name="c", subcore_axis_name="s",
                                  num_cores=NC, num_subcores=NS)
    f32 = jnp.float32
    kern = pl.kernel(
        _sc_body,
        out_type=[
            jax.ShapeDtypeStruct((NQ * 2 * N, DH), f32),
            jax.ShapeDtypeStruct((2 * N,), f32),
            jax.ShapeDtypeStruct((2 * N,), f32),
            jax.ShapeDtypeStruct((2 * N,), f32),
        ],
        mesh=mesh,
        compiler_params=pltpu.CompilerParams(needs_layout_passes=False,
                                             use_tc_tiling_on_sc=False),
        scratch_types=[
            pltpu.VMEM((N,), f32),
            pltpu.VMEM((N,), f32),
            pltpu.VMEM((EPT,), f32),
            pltpu.VMEM((2, SUP, CH), jnp.int32),
            pltpu.VMEM((2, SUP, CH), jnp.int32),
            pltpu.VMEM((CH, DH), f32),
            pltpu.VMEM((CH, DH), f32),
            pltpu.VMEM((CH, DH), f32),
            pltpu.VMEM((CH, DH), f32),
            pltpu.VMEM((CH, DH), f32),
            pltpu.VMEM((NPAD,), f32),
            pltpu.VMEM((NPAD,), f32),
            pltpu.VMEM((NPAD,), f32),
            pltpu.VMEM((NPAD,), f32),
            pltpu.VMEM((NS, RED_W), f32),
            pltpu.VMEM_SHARED((N, DH), f32),
            pltpu.VMEM_SHARED((NS, NPAD2), f32),
            pltpu.SemaphoreType.DMA,
            pltpu.SemaphoreType.DMA,
            pltpu.SemaphoreType.DMA,
            pltpu.SemaphoreType.DMA,
            pltpu.SemaphoreType.DMA,
            pltpu.SemaphoreType.DMA,
            pltpu.SemaphoreType.DMA,
            pltpu.SemaphoreType.DMA,
            pltpu.SemaphoreType.DMA,
            pltpu.SemaphoreType.DMA,
        ],
    )
    return kern(src2, dst2, h_all, as_cat, ad_cat)


def _final_body(*refs):
    fq0 = refs[0:NQ]
    fq1 = refs[NQ:2 * NQ]
    dn0_ref, dn1_ref, ss0_ref, ss1_ref, cnt_ref = refs[2 * NQ:2 * NQ + 5]
    feat_ref, score_ref = refs[2 * NQ + 5:]
    d0 = dn0_ref[...] + EPS
    d1 = dn1_ref[...] + EPS
    for q in range(NQ):
        f = fq0[q][...] / d0
        g = fq1[q][...] / d1
        feat_ref[:, pl.ds(q * DH, DH)] = (jnp.maximum(f, 0.01 * f)
                                          + jnp.maximum(g, 0.01 * g))
    ssum = ss0_ref[...] + ss1_ref[...]
    score_ref[...] = ssum / jnp.maximum(cnt_ref[...], 1.0)


def _final_stage(fq0, fq1, dn0, dn1, ss0, ss1, cntv):
    blk = 1000
    grid = N // blk
    half_spec = pl.BlockSpec((blk, DH), lambda i: (i, 0))
    col_spec = pl.BlockSpec((blk, 1), lambda i: (i, 0))
    return pl.pallas_call(
        _final_body,
        grid=(grid,),
        in_specs=[half_spec] * (2 * NQ) + [col_spec] * 5,
        out_specs=[pl.BlockSpec((blk, D), lambda i: (i, 0)), col_spec],
        out_shape=[
            jax.ShapeDtypeStruct((N, D), jnp.float32),
            jax.ShapeDtypeStruct((N, 1), jnp.float32),
        ],
    )(*fq0, *fq1, dn0, dn1, ss0, ss1, cntv)


@jax.jit
def kernel(x, edge_index, W0, att_src0, att_dst0, W1, att_src1, att_dst1):
    h_all, as_cat, ad_cat = _dense_stage(x, W0, att_src0, att_dst0,
                                         W1, att_src1, att_dst1)
    src2 = edge_index[0].reshape(E // CH, CH)
    dst2 = edge_index[1].reshape(E // CH, CH)
    sc_outs = _sc_stage(src2, dst2, h_all.reshape(NQ * 2 * N, DH),
                        as_cat.reshape(2 * N), ad_cat.reshape(2 * N))
    feat_all, denom, ssum, cnt = sc_outs
    fq0 = [feat_all[q * 2 * N:q * 2 * N + N] for q in range(NQ)]
    fq1 = [feat_all[q * 2 * N + N:(q + 1) * 2 * N] for q in range(NQ)]
    feature, score = _final_stage(
        fq0, fq1,
        denom[:N].reshape(N, 1), denom[N:].reshape(N, 1),
        ssum[:N].reshape(N, 1), ssum[N:].reshape(N, 1),
        cnt[:N].reshape(N, 1))
    return feature, score.reshape(N)


# speedup vs baseline: 31.4300x; 1.0057x over previous
import jax
import jax.numpy as jnp
from jax import lax
from jax.experimental import pallas as pl
from jax.experimental.pallas import tpu as pltpu
from jax.experimental.pallas import tpu_sc as plsc

N = 10000
D = 128
DH = 32
NQ = D // DH
E = 320000
NC = 2
NS = 16
L = 16

EPT = E // NS
CH = 80
NCHUNK = EPT // CH
SUP = 5
PAIR = 2 * SUP
NITER = NCHUNK // PAIR
NBUF = 4
ROWS_MAIN = 632
ROWS_LAST = N - (NS - 1) * ROWS_MAIN
RED_W = 640
NPAD = NS * RED_W
NPAD2 = NPAD // 2
RED_LAST = N - (NS - 1) * RED_W
EPS = 1e-16


def _dense_body(x_ref, w0_ref, s0_ref, d0_ref, w1_ref, s1_ref, d1_ref,
                h_ref, as_ref, ad_ref):
    x = x_ref[...]
    h0 = jnp.dot(x, w0_ref[...], preferred_element_type=jnp.float32)
    h1 = jnp.dot(x, w1_ref[...], preferred_element_type=jnp.float32)
    for q in range(NQ):
        h_ref[q, 0] = h0[:, q * DH:(q + 1) * DH]
        h_ref[q, 1] = h1[:, q * DH:(q + 1) * DH]
    as_ref[0] = jnp.sum(h0 * s0_ref[...][None, :], axis=1, keepdims=True)
    as_ref[1] = jnp.sum(h1 * s1_ref[...][None, :], axis=1, keepdims=True)
    ad_ref[0] = jnp.sum(h0 * d0_ref[...][None, :], axis=1, keepdims=True)
    ad_ref[1] = jnp.sum(h1 * d1_ref[...][None, :], axis=1, keepdims=True)


def _dense_stage(x, W0, a_s0, a_d0, W1, a_s1, a_d1):
    blk = 2000
    grid = N // blk
    full = lambda i: (0, 0)
    return pl.pallas_call(
        _dense_body,
        grid=(grid,),
        in_specs=[pl.BlockSpec((blk, D), lambda i: (i, 0))]
        + [pl.BlockSpec((D, D), full), pl.BlockSpec((D,), lambda i: (0,)),
           pl.BlockSpec((D,), lambda i: (0,))] * 2,
        out_specs=[pl.BlockSpec((NQ, 2, blk, DH), lambda i: (0, 0, i, 0))]
        + [pl.BlockSpec((2, blk, 1), lambda i: (0, i, 0))] * 2,
        out_shape=[jax.ShapeDtypeStruct((NQ, 2, N, DH), jnp.float32),
                   jax.ShapeDtypeStruct((2, N, 1), jnp.float32),
                   jax.ShapeDtypeStruct((2, N, 1), jnp.float32)],
    )(x, W0, a_s0, a_d0, W1, a_s1, a_d1)


def _sc_body(src2_hbm, dst2_hbm, h_hbm, as_hbm, ad_hbm,
             feat_out, denom, ssum, cnt,
             asv, adv, pall, sidx, didx, sdix,
             rows0, rows1, rows2, rows3, zbuf,
             denp, cntp, scop, dnf, redbuf,
             feat_sh, parts_sh,
             gs0, gs1, gs2, gs3, ss0, ss1, ss2, ss3, is0, is1):
    rows_bufs = (rows0, rows1, rows2, rows3)
    gsems = (gs0, gs1, gs2, gs3)
    ssems = (ss0, ss1, ss2, ss3)
    isems = (is0, is1)
    c = lax.axis_index("c")
    s = lax.axis_index("s")
    cN = c * N
    chunk0 = s * NCHUNK
    zeros16 = jnp.zeros((L,), jnp.float32)
    ones16 = jnp.ones((L,), jnp.float32)

    pltpu.sync_copy(as_hbm.at[pl.ds(cN, N)], asv)
    pltpu.sync_copy(ad_hbm.at[pl.ds(cN, N)], adv)

    @pl.loop(0, CH)
    def _(i):
        for k in range(DH // L):
            zbuf[i, pl.ds(k * L, L)] = zeros16

    @pl.loop(0, NPAD // L)
    def _(i):
        denp[pl.ds(i * L, L)] = zeros16
        cntp[pl.ds(i * L, L)] = zeros16
        scop[pl.ds(i * L, L)] = zeros16

    zeros16i = jnp.zeros((L,), jnp.int32)
    for b in range(NBUF):
        for i in range(CH // L):
            sdix[b, pl.ds(i * L, L)] = zeros16i

    row0 = s * ROWS_MAIN

    def zero_feat_rows(nrows):
        off = 0
        while nrows - off >= CH:
            pltpu.sync_copy(zbuf.at[pl.ds(0, CH)],
                            feat_sh.at[pl.ds(row0 + off, CH)])
            off += CH
        if nrows - off:
            pltpu.sync_copy(zbuf.at[pl.ds(0, nrows - off)],
                            feat_sh.at[pl.ds(row0 + off, nrows - off)])

    def zero_feat():
        @pl.when(s < NS - 1)
        def _():
            zero_feat_rows(ROWS_MAIN)

        @pl.when(s == NS - 1)
        def _():
            zero_feat_rows(ROWS_LAST)

    def dump_feat(qd):
        obase = qd * (2 * N) + cN + row0

        @pl.when(s < NS - 1)
        def _():
            pltpu.sync_copy(feat_sh.at[pl.ds(row0, ROWS_MAIN)],
                            feat_out.at[pl.ds(obase, ROWS_MAIN)])

        @pl.when(s == NS - 1)
        def _():
            pltpu.sync_copy(feat_sh.at[pl.ds(row0, ROWS_LAST)],
                            feat_out.at[pl.ds(obase, ROWS_LAST)])

    def scale_rows(jj, rows):
        @pl.loop(0, CH // L)
        def _(g):
            p16 = pall[pl.ds(jj * CH + g * L, L)]
            for lane in range(L):
                ps = p16[lane]
                e = g * L + lane
                for k in range(DH // L):
                    rows[e, pl.ds(k * L, L)] = rows[e, pl.ds(k * L, L)] * ps

    def reduce_all(buf, hbm_out):
        colbase = s * RED_W

        for r in range(2):
            pltpu.sync_copy(buf.at[pl.ds(r * NPAD2, NPAD2)],
                            parts_sh.at[s])
            plsc.subcore_barrier()

            @pl.when((s // 8) == r)
            def _(r=r):
                pltpu.sync_copy(
                    parts_sh.at[:, pl.ds((s % 8) * RED_W, RED_W)], redbuf)

                @pl.loop(0, RED_W // L)
                def _(g):
                    acc = redbuf[0, pl.ds(g * L, L)]
                    for t in range(1, NS):
                        acc = acc + redbuf[t, pl.ds(g * L, L)]
                    buf[pl.ds(colbase + g * L, L)] = acc

            plsc.subcore_barrier()

        @pl.when(s < NS - 1)
        def _():
            pltpu.sync_copy(buf.at[pl.ds(colbase, RED_W)],
                            hbm_out.at[pl.ds(cN + colbase, RED_W)])

        @pl.when(s == NS - 1)
        def _():
            pltpu.sync_copy(buf.at[pl.ds(colbase, RED_LAST)],
                            hbm_out.at[pl.ds(cN + colbase, RED_LAST)])

    @pl.loop(0, NQ)
    def _(qd):
        @pl.when(qd == 1)
        def _():
            reduce_all(denp, denom)
            plsc.subcore_barrier()
            pltpu.sync_copy(denom.at[pl.ds(cN, N)], dnf.at[pl.ds(0, N)])

        zero_feat()
        plsc.subcore_barrier()

        hbase = qd * (2 * N) + cN

        def issue_idx(buf, rowoff):
            pltpu.async_copy(src2_hbm.at[pl.ds(rowoff, SUP)],
                             sidx.at[buf], isems[buf])
            pltpu.async_copy(dst2_hbm.at[pl.ds(rowoff, SUP)],
                             didx.at[buf], isems[buf])

        def wait_idx(buf):
            for _ in range(2):
                pltpu.make_async_copy(src2_hbm.at[pl.ds(chunk0, SUP)],
                                      sidx.at[buf], isems[buf]).wait()

        for b in range(NBUF):
            pltpu.async_copy(zbuf, feat_sh.at[sdix.at[0]], ssems[b],
                             add=True)

        issue_idx(0, chunk0)
        issue_idx(1, chunk0 + SUP)

        @pl.loop(0, NITER)
        def _(u):
            jbase = u * PAIR
            nextoff = chunk0 + ((u + 1) % NITER) * PAIR
            gdescs = [None] * NBUF

            for step in range(PAIR + 2):
                k = step
                if k < PAIR:
                    half, kk = divmod(k, SUP)
                    if kk == 0:
                        wait_idx(half)
                    for i in range(CH // L):
                        sv = sidx[half, kk, pl.ds(i * L, L)]
                        dv = didx[half, kk, pl.ds(i * L, L)]

                        @pl.when(qd == 0)
                        def _(sv=sv, dv=dv, k=k, i=i):
                            a = (plsc.load_gather(asv, [sv])
                                 + plsc.load_gather(adv, [dv]))
                            a = jnp.maximum(a, 0.2 * a)
                            p = jnp.exp(a)
                            pall[pl.ds((jbase + k) * CH + i * L, L)] = p
                            plsc.addupdate_scatter(denp, [dv], p)
                            plsc.addupdate_scatter(cntp, [sv], ones16)

                        @pl.when(qd == 1)
                        def _(sv=sv, dv=dv, k=k, i=i):
                            p = pall[pl.ds((jbase + k) * CH + i * L, L)]
                            dn = plsc.load_gather(dnf, [dv])
                            plsc.addupdate_scatter(scop, [sv],
                                                   p / (dn + EPS))

                        sidx[half, kk, pl.ds(i * L, L)] = sv + hbase
                    b = k % NBUF
                    pltpu.make_async_copy(rows_bufs[b],
                                          feat_sh.at[sdix.at[0]],
                                          ssems[b]).wait()
                    gdescs[b] = pltpu.async_copy(
                        h_hbm.at[sidx.at[half, kk]], rows_bufs[b],
                        gsems[b])
                if 0 <= k - 2 < PAIR:
                    h2, kk2 = divmod(k - 2, SUP)
                    b2 = (k - 2) % NBUF
                    gdescs[b2].wait()
                    for i in range(CH // L):
                        sdix[b2, pl.ds(i * L, L)] = \
                            didx[h2, kk2, pl.ds(i * L, L)]
                    scale_rows(jbase + (k - 2), rows_bufs[b2])
                    pltpu.async_copy(rows_bufs[b2],
                                     feat_sh.at[sdix.at[b2]],
                                     ssems[b2], add=True)
                    if k - 2 == SUP - 1:
                        issue_idx(0, nextoff)

            issue_idx(1, nextoff + SUP)

        wait_idx(0)
        wait_idx(1)
        for b in range(NBUF):
            pltpu.make_async_copy(rows_bufs[b], feat_sh.at[sdix.at[0]],
                                  ssems[b]).wait()

        plsc.subcore_barrier()
        dump_feat(qd)
        plsc.subcore_barrier()

    reduce_all(scop, ssum)
    reduce_all(cntp, cnt)


def _sc_stage(src2, dst2, h_all, as_cat, ad_cat):
    mesh = plsc.VectorSubcoreMesh(core_axis_name="c", subcore_axis_name="s",
                                  num_cores=NC, num_subcores=NS)
    f32 = jnp.float32
    kern = pl.kernel(
        _sc_body,
        out_type=[
            jax.ShapeDtypeStruct((NQ * 2 * N, DH), f32),
            jax.ShapeDtypeStruct((2 * N,), f32),
            jax.ShapeDtypeStruct((2 * N,), f32),
            jax.ShapeDtypeStruct((2 * N,), f32),
        ],
        mesh=mesh,
        compiler_params=pltpu.CompilerParams(needs_layout_passes=False,
                                             use_tc_tiling_on_sc=False),
        scratch_types=[
            pltpu.VMEM((N,), f32),
            pltpu.VMEM((N,), f32),
            pltpu.VMEM((EPT,), f32),
            pltpu.VMEM((2, SUP, CH), jnp.int32),
            pltpu.VMEM((2, SUP, CH), jnp.int32),
            pltpu.VMEM((NBUF, CH), jnp.int32),
            pltpu.VMEM((CH, DH), f32),
            pltpu.VMEM((CH, DH), f32),
            pltpu.VMEM((CH, DH), f32),
            pltpu.VMEM((CH, DH), f32),
            pltpu.VMEM((CH, DH), f32),
            pltpu.VMEM((NPAD,), f32),
            pltpu.VMEM((NPAD,), f32),
            pltpu.VMEM((NPAD,), f32),
            pltpu.VMEM((NPAD,), f32),
            pltpu.VMEM((NS, RED_W), f32),
            pltpu.VMEM_SHARED((N, DH), f32),
            pltpu.VMEM_SHARED((NS, NPAD2), f32),
            pltpu.SemaphoreType.DMA,
            pltpu.SemaphoreType.DMA,
            pltpu.SemaphoreType.DMA,
            pltpu.SemaphoreType.DMA,
            pltpu.SemaphoreType.DMA,
            pltpu.SemaphoreType.DMA,
            pltpu.SemaphoreType.DMA,
            pltpu.SemaphoreType.DMA,
            pltpu.SemaphoreType.DMA,
            pltpu.SemaphoreType.DMA,
        ],
    )
    return kern(src2, dst2, h_all, as_cat, ad_cat)


def _final_body(*refs):
    fq0 = refs[0:NQ]
    fq1 = refs[NQ:2 * NQ]
    dn0_ref, dn1_ref, ss0_ref, ss1_ref, cnt_ref = refs[2 * NQ:2 * NQ + 5]
    feat_ref, score_ref = refs[2 * NQ + 5:]
    d0 = dn0_ref[...] + EPS
    d1 = dn1_ref[...] + EPS
    for q in range(NQ):
        f = fq0[q][...] / d0
        g = fq1[q][...] / d1
        feat_ref[:, pl.ds(q * DH, DH)] = (jnp.maximum(f, 0.01 * f)
                                          + jnp.maximum(g, 0.01 * g))
    ssum = ss0_ref[...] + ss1_ref[...]
    score_ref[...] = ssum / jnp.maximum(cnt_ref[...], 1.0)


def _final_stage(fq0, fq1, dn0, dn1, ss0, ss1, cntv):
    blk = 1000
    grid = N // blk
    half_spec = pl.BlockSpec((blk, DH), lambda i: (i, 0))
    col_spec = pl.BlockSpec((blk, 1), lambda i: (i, 0))
    return pl.pallas_call(
        _final_body,
        grid=(grid,),
        in_specs=[half_spec] * (2 * NQ) + [col_spec] * 5,
        out_specs=[pl.BlockSpec((blk, D), lambda i: (i, 0)), col_spec],
        out_shape=[
            jax.ShapeDtypeStruct((N, D), jnp.float32),
            jax.ShapeDtypeStruct((N, 1), jnp.float32),
        ],
    )(*fq0, *fq1, dn0, dn1, ss0, ss1, cntv)


@jax.jit
def kernel(x, edge_index, W0, att_src0, att_dst0, W1, att_src1, att_dst1):
    h_all, as_cat, ad_cat = _dense_stage(x, W0, att_src0, att_dst0,
                                         W1, att_src1, att_dst1)
    src2 = edge_index[0].reshape(E // CH, CH)
    dst2 = edge_index[1].reshape(E // CH, CH)
    sc_outs = _sc_stage(src2, dst2, h_all.reshape(NQ * 2 * N, DH),
                        as_cat.reshape(2 * N), ad_cat.reshape(2 * N))
    feat_all, denom, ssum, cnt = sc_outs
    fq0 = [feat_all[q * 2 * N:q * 2 * N + N] for q in range(NQ)]
    fq1 = [feat_all[q * 2 * N + N:(q + 1) * 2 * N] for q in range(NQ)]
    feature, score = _final_stage(
        fq0, fq1,
        denom[:N].reshape(N, 1), denom[N:].reshape(N, 1),
        ssum[:N].reshape(N, 1), ssum[N:].reshape(N, 1),
        cnt[:N].reshape(N, 1))
    return feature, score.reshape(N)
